# baseline (speedup 1.0000x reference)
import jax
import jax.numpy as jnp
from jax import lax
from jax.experimental import pallas as pl
from jax.experimental.pallas import tpu as pltpu
from jax.experimental.pallas import tpu_sc as plsc

B = 16384
D = 64
NC = 2
NS = 16
NW = NC * NS
BPW = B // NW
GCH = 128
NGCH = BPW // GCH

_SCR = 17 * 16

_MESH = plsc.VectorSubcoreMesh(
    core_axis_name="c", subcore_axis_name="s",
    num_cores=NC, num_subcores=NS)

_PARAMS = pltpu.CompilerParams(
    needs_layout_passes=False, use_tc_tiling_on_sc=False)


def _wid_base():
  wid = lax.axis_index("s") * NC + lax.axis_index("c")
  return wid * BPW


def _body_a(item_p_hbm, item_n_hbm, ifl, pn_out,
            idx_p, idx_n, p_rows, n_rows, sem_p, sem_n):
  base = _wid_base()
  for j in range(NGCH):
    pltpu.sync_copy(item_p_hbm.at[pl.ds(base + j * GCH, GCH)], idx_p.at[j])
    pltpu.sync_copy(item_n_hbm.at[pl.ds(base + j * GCH, GCH)], idx_n.at[j])
  copies = []
  for j in range(NGCH):
    sl = pl.ds(j * GCH, GCH)
    copies.append(pltpu.async_copy(ifl.at[idx_p.at[j]], p_rows.at[sl], sem_p))
    copies.append(pltpu.async_copy(ifl.at[idx_n.at[j]], n_rows.at[sl], sem_n))
  for c in copies:
    c.wait()

  def sub(g, carry):
    r0 = g * 16
    for j in range(16):
      for q in range(D // 16):
        sl = pl.ds(q * 16, 16)
        p_rows[r0 + j, sl] = p_rows[r0 + j, sl] - n_rows[r0 + j, sl]
    return carry

  lax.fori_loop(0, BPW // 16, sub, 0)
  pltpu.sync_copy(p_rows, pn_out.at[pl.ds(base, BPW)])


def _body_b(user_hbm, ufl, pn_hbm, out_hbm,
            idx_u, u_rows, pn_v, out_v, scr, sem_u, sem_pn):
  base = _wid_base()
  for j in range(NGCH):
    pltpu.sync_copy(user_hbm.at[pl.ds(base + j * GCH, GCH)], idx_u.at[j])
  copies = [pltpu.async_copy(
      pn_hbm.at[pl.ds(base, BPW)], pn_v, sem_pn)]
  for j in range(NGCH):
    sl = pl.ds(j * GCH, GCH)
    copies.append(pltpu.async_copy(ufl.at[idx_u.at[j]], u_rows.at[sl], sem_u))
  for c in copies:
    c.wait()

  lane = lax.iota(jnp.int32, 16)
  lane17 = lane * 17

  def grp(g, carry):
    g0 = g * 16
    for j in range(16):
      r = g0 + j
      acc = None
      for q in range(D // 16):
        sl = pl.ds(q * 16, 16)
        t = u_rows[r, sl] * pn_v[r, sl]
        acc = t if acc is None else acc + t
      plsc.store_scatter(scr, [lane17 + j], acc)
    tot = None
    for d in range(16):
      v = scr[pl.ds(d * 17, 16)]
      tot = v if tot is None else tot + v
    out_v[pl.ds(g * 16, 16)] = tot
    return carry

  lax.fori_loop(0, BPW // 16, grp, 0)
  pltpu.sync_copy(out_v, out_hbm.at[pl.ds(base, BPW)])


@jax.jit
def kernel(user, item_p, item_n, user_factors, item_factors):
  uf_lin = lax.optimization_barrier(user_factors.reshape(-1)).reshape(
      user_factors.shape[0], D)
  if_lin = lax.optimization_barrier(item_factors.reshape(-1)).reshape(
      item_factors.shape[0], D)

  a = pl.kernel(
      _body_a,
      out_type=jax.ShapeDtypeStruct((B, D), jnp.float32),
      mesh=_MESH,
      compiler_params=_PARAMS,
      scratch_types=[
          pltpu.VMEM((NGCH, GCH), jnp.int32),
          pltpu.VMEM((NGCH, GCH), jnp.int32),
          pltpu.VMEM((BPW, D), jnp.float32),
          pltpu.VMEM((BPW, D), jnp.float32),
          pltpu.SemaphoreType.DMA,
          pltpu.SemaphoreType.DMA,
      ],
  )
  pn = a(item_p, item_n, if_lin)

  b = pl.kernel(
      _body_b,
      out_type=jax.ShapeDtypeStruct((B,), jnp.float32),
      mesh=_MESH,
      compiler_params=_PARAMS,
      scratch_types=[
          pltpu.VMEM((NGCH, GCH), jnp.int32),
          pltpu.VMEM((BPW, D), jnp.float32),
          pltpu.VMEM((BPW, D), jnp.float32),
          pltpu.VMEM((BPW,), jnp.float32),
          pltpu.VMEM((_SCR,), jnp.float32),
          pltpu.SemaphoreType.DMA,
          pltpu.SemaphoreType.DMA,
      ],
  )
  return b(user, uf_lin, pn)
